# Initial kernel scaffold; baseline (speedup 1.0000x reference)
import jax
import jax.numpy as jnp
from jax.experimental import pallas as pl


def kernel(indices, weight):
    raise NotImplementedError("write your pallas kernel here")


# speedup vs baseline: 1.3038x; 1.3038x over previous
import functools

import jax
from jax import lax
import jax.numpy as jnp
from jax.experimental import pallas as pl
from jax.experimental.pallas import tpu as pltpu
from jax.experimental.pallas import tpu_sc as plsc

_NUM_ROWS = 1000
_D = 4096
_B = 16384
_NC = 2
_NS = 16
_NW = _NC * _NS
_BPW = _B // _NW
_C = 8
_NCHUNK = _BPW // _C


def kernel(indices, weight):
    flat = weight.reshape(_NUM_ROWS, _D)
    idx = indices.astype(jnp.int32)
    mesh = plsc.VectorSubcoreMesh(
        core_axis_name="core", subcore_axis_name="subcore"
    )

    @functools.partial(
        pl.kernel,
        out_type=jax.ShapeDtypeStruct((_B, _D), jnp.float32),
        mesh=mesh,
        scratch_types=[
            pltpu.VMEM((_BPW,), jnp.int32),
            pltpu.VMEM((_C, _D), jnp.float32),
            pltpu.VMEM((_C, _D), jnp.float32),
            pltpu.SemaphoreType.DMA,
            pltpu.SemaphoreType.DMA,
        ],
    )
    def gather_kernel(x_hbm, i_hbm, o_hbm, idx_v, buf0, buf1, sem0, sem1):
        wid = lax.axis_index("subcore") * _NC + lax.axis_index("core")
        base = wid * _BPW
        pltpu.sync_copy(i_hbm.at[pl.ds(base, _BPW)], idx_v)

        bufs = (buf0, buf1)
        sems = (sem0, sem1)

        def start_gather(g, b):
            pltpu.async_copy(
                x_hbm.at[idx_v.at[pl.ds(g * _C, _C)]], bufs[b], sems[b]
            )

        start_gather(0, 0)

        @pl.loop(0, _NCHUNK, step=2)
        def _(g0):
            for b in range(2):
                g = g0 + b
                pltpu.make_async_copy(
                    x_hbm.at[idx_v.at[pl.ds(g * _C, _C)]], bufs[b], sems[b]
                ).wait()

                @pl.when(g + 1 < _NCHUNK)
                def _():
                    start_gather(g + 1, 1 - b)

                pltpu.sync_copy(
                    bufs[b], o_hbm.at[pl.ds(base + g * _C, _C)]
                )

    out = gather_kernel(flat, idx)
    return out.reshape(_B, 64, 64)


# speedup vs baseline: 1.3225x; 1.0144x over previous
import functools

import jax
from jax import lax
import jax.numpy as jnp
from jax.experimental import pallas as pl
from jax.experimental.pallas import tpu as pltpu
from jax.experimental.pallas import tpu_sc as plsc

_NUM_ROWS = 1000
_D = 4096
_B = 16384
_NC = 2
_NS = 16
_NW = _NC * _NS
_BPW = _B // _NW
_C = 8
_NCHUNK = _BPW // _C
_NBUF = 3


def kernel(indices, weight):
    flat = weight.reshape(_NUM_ROWS, _D)
    idx = indices.astype(jnp.int32)
    mesh = plsc.VectorSubcoreMesh(
        core_axis_name="core", subcore_axis_name="subcore"
    )

    @functools.partial(
        pl.kernel,
        out_type=jax.ShapeDtypeStruct((_B, _D), jnp.float32),
        mesh=mesh,
        scratch_types=[
            pltpu.VMEM((_BPW,), jnp.int32),
            pltpu.VMEM((_C, _D), jnp.float32),
            pltpu.VMEM((_C, _D), jnp.float32),
            pltpu.VMEM((_C, _D), jnp.float32),
            pltpu.SemaphoreType.DMA,
            pltpu.SemaphoreType.DMA,
            pltpu.SemaphoreType.DMA,
            pltpu.SemaphoreType.DMA,
            pltpu.SemaphoreType.DMA,
            pltpu.SemaphoreType.DMA,
        ],
    )
    def gather_kernel(
        x_hbm, i_hbm, o_hbm, idx_v,
        buf0, buf1, buf2, gsem0, gsem1, gsem2, ssem0, ssem1, ssem2,
    ):
        wid = lax.axis_index("subcore") * _NC + lax.axis_index("core")
        base = wid * _BPW
        pltpu.sync_copy(i_hbm.at[pl.ds(base, _BPW)], idx_v)

        bufs = (buf0, buf1, buf2)
        gsems = (gsem0, gsem1, gsem2)
        ssems = (ssem0, ssem1, ssem2)

        def gather_copy(g, j):
            return pltpu.make_async_copy(
                x_hbm.at[idx_v.at[pl.ds(g * _C, _C)]], bufs[j], gsems[j]
            )

        def store_copy(g, j):
            return pltpu.make_async_copy(
                bufs[j], o_hbm.at[pl.ds(base + g * _C, _C)], ssems[j]
            )

        gather_copy(0, 0).start()

        @pl.loop(0, _NCHUNK + _NBUF - 1, step=_NBUF)
        def _(g0):
            for b in range(_NBUF):
                g = g0 + b
                jn = (b + 1) % _NBUF

                @pl.when(g < _NCHUNK)
                def _():
                    @pl.when(jnp.logical_and(g + 1 >= _NBUF, g + 1 < _NCHUNK))
                    def _():
                        store_copy(g + 1 - _NBUF, jn).wait()

                    @pl.when(g + 1 < _NCHUNK)
                    def _():
                        gather_copy(g + 1, jn).start()

                    gather_copy(g, b).wait()
                    store_copy(g, b).start()

        for g in range(_NCHUNK - _NBUF, _NCHUNK):
            store_copy(g, g % _NBUF).wait()

    out = gather_kernel(flat, idx)
    return out.reshape(_B, 64, 64)


# speedup vs baseline: 1.3228x; 1.0003x over previous
import functools

import jax
from jax import lax
import jax.numpy as jnp
from jax.experimental import pallas as pl
from jax.experimental.pallas import tpu as pltpu
from jax.experimental.pallas import tpu_sc as plsc

_NUM_ROWS = 1000
_D = 4096
_B = 16384
_NC = 2
_NS = 16
_NW = _NC * _NS
_BPW = _B // _NW
_C = 8
_NCHUNK = _BPW // _C
_NBUF = 3
_GAHEAD = 2


def kernel(indices, weight):
    flat = weight.reshape(_NUM_ROWS, _D)
    idx = indices.astype(jnp.int32)
    mesh = plsc.VectorSubcoreMesh(
        core_axis_name="core", subcore_axis_name="subcore"
    )

    @functools.partial(
        pl.kernel,
        out_type=jax.ShapeDtypeStruct((_B, _D), jnp.float32),
        mesh=mesh,
        scratch_types=[
            pltpu.VMEM((_BPW,), jnp.int32),
            pltpu.VMEM((_C, _D), jnp.float32),
            pltpu.VMEM((_C, _D), jnp.float32),
            pltpu.VMEM((_C, _D), jnp.float32),
            pltpu.SemaphoreType.DMA,
            pltpu.SemaphoreType.DMA,
            pltpu.SemaphoreType.DMA,
            pltpu.SemaphoreType.DMA,
            pltpu.SemaphoreType.DMA,
            pltpu.SemaphoreType.DMA,
        ],
    )
    def gather_kernel(
        x_hbm, i_hbm, o_hbm, idx_v,
        buf0, buf1, buf2,
        gsem0, gsem1, gsem2, ssem0, ssem1, ssem2,
    ):
        wid = lax.axis_index("subcore") * _NC + lax.axis_index("core")
        base = wid * _BPW
        pltpu.sync_copy(i_hbm.at[pl.ds(base, _BPW)], idx_v)

        bufs = (buf0, buf1, buf2)
        gsems = (gsem0, gsem1, gsem2)
        ssems = (ssem0, ssem1, ssem2)

        def gather_copy(g, j):
            return pltpu.make_async_copy(
                x_hbm.at[idx_v.at[pl.ds(g * _C, _C)]], bufs[j], gsems[j]
            )

        def store_copy(g, j):
            return pltpu.make_async_copy(
                bufs[j], o_hbm.at[pl.ds(base + g * _C, _C)], ssems[j]
            )

        for g in range(_GAHEAD):
            gather_copy(g, g).start()

        @pl.loop(0, _NCHUNK + (-_NCHUNK) % _NBUF, step=_NBUF)
        def _(g0):
            for b in range(_NBUF):
                g = g0 + b
                jn = (b + _GAHEAD) % _NBUF

                @pl.when(g < _NCHUNK)
                def _():
                    @pl.when(g + _GAHEAD < _NCHUNK)
                    def _():
                        @pl.when(g >= _NBUF - _GAHEAD)
                        def _():
                            store_copy(g - (_NBUF - _GAHEAD), jn).wait()

                        gather_copy(g + _GAHEAD, jn).start()

                    gather_copy(g, b).wait()
                    store_copy(g, b).start()

        for g in range(_NCHUNK - _NBUF, _NCHUNK):
            store_copy(g, g % _NBUF).wait()

    out = gather_kernel(flat, idx)
    return out.reshape(_B, 64, 64)
